# Initial kernel scaffold; baseline (speedup 1.0000x reference)
import jax
import jax.numpy as jnp
from jax.experimental import pallas as pl


def kernel(suppressants, capacity, equipment, refilled_suppressants, randomness_source, equipment_bonuses):
    raise NotImplementedError("write your pallas kernel here")


# speedup vs baseline: 3.7983x; 3.7983x over previous
import functools

import jax
import jax.numpy as jnp
from jax.experimental import pallas as pl
from jax.experimental.pallas import tpu as pltpu

_REFILL_PROBABILITY = 0.5


def _body(bon_ref, sup_ref, cap_ref, eq_ref, refil_ref, rand_ref, out_ref):
    mask = jnp.logical_and(refil_ref[...], rand_ref[...] < _REFILL_PROBABILITY)
    eq = eq_ref[...]
    b0 = bon_ref[0]
    b1 = bon_ref[1]
    b2 = bon_ref[2]
    bon = jnp.where(eq == 1, b1, jnp.where(eq == 2, b2, b0))
    out_ref[...] = jnp.where(mask, cap_ref[...] + bon, sup_ref[...])


def kernel(suppressants, capacity, equipment, refilled_suppressants,
           randomness_source, equipment_bonuses):
    B, A = suppressants.shape
    block_b = 512
    grid = (B // block_b,)
    blk = lambda: pl.BlockSpec((block_b, A), lambda i: (i, 0))
    return pl.pallas_call(
        _body,
        grid=grid,
        in_specs=[
            pl.BlockSpec(memory_space=pltpu.SMEM),
            blk(), blk(), blk(), blk(), blk(),
        ],
        out_specs=blk(),
        out_shape=jax.ShapeDtypeStruct((B, A), jnp.float32),
        compiler_params=pltpu.CompilerParams(
            dimension_semantics=("arbitrary",),
        ),
    )(equipment_bonuses, suppressants, capacity, equipment,
      refilled_suppressants, randomness_source)
